# baseline (speedup 1.0000x reference)
import functools

import jax
import jax.numpy as jnp
from jax import lax
from jax.experimental import pallas as pl
from jax.experimental.pallas import tpu as pltpu
from jax.experimental.pallas import tpu_sc as plsc

_TOTAL_AMOUNT = 30000000.0
_MIN_AMOUNT = 1500.0
_MIN_RATIO = _MIN_AMOUNT / _TOTAL_AMOUNT
_N = 65536
_NVREG = _N // 16
_NBIN = 1024
_R0 = 32768
_POS_INF_BITS = 0x7F800000
_CAP = 256
_U = 8
_NCHUNK = 8
_CHUNK = _N // _NCHUNK


def _body(x0, x1, x2, x3, x4, x5, x6, x7, x8, x9, x10, x11, x12, x13, x14,
          x15, out_ref, inter_ref, xv, hist, candv, rowv, bufv, outv, sems):
  xs = (x0, x1, x2, x3, x4, x5, x6, x7, x8, x9, x10, x11, x12, x13, x14, x15)
  c = lax.axis_index("c")
  s = lax.axis_index("s")
  lane = lax.iota(jnp.int32, 16)
  ones = jnp.ones((16,), jnp.int32)
  zeros = jnp.zeros((16,), jnp.int32)
  inf16 = jnp.full((16,), _POS_INF_BITS, jnp.int32)

  @pl.when(c == 0)
  def _core0():
    for a in range(16):
      @pl.when(s == a)
      def _load():
        for j in range(_NCHUNK):
          pltpu.make_async_copy(
              xs[a].at[pl.ds(j * _CHUNK, _CHUNK)],
              xv.at[pl.ds(j * _CHUNK, _CHUNK)],
              sems.at[j]).start()

    def zero_hist():
      @plsc.parallel_loop(0, _NBIN * 16, step=16, unroll=_U)
      def _zb(i):
        hist[pl.ds(i, 16)] = zeros

    def hist_chunk(j, shift):
      @plsc.parallel_loop(j * _CHUNK, (j + 1) * _CHUNK, step=16, unroll=_U)
      def _pb(i):
        v = xv[pl.ds(i, 16)]
        k = plsc.bitcast(v, jnp.int32)
        b = (k >> shift) & (_NBIN - 1)
        plsc.addupdate_scatter(hist, [b * 16 + lane], ones)

    def hist_pass_dyn(shift, match_shift, match_prefix, active):
      def zb(i, carry):
        base = i * 16 * _U
        for u in range(_U):
          hist[pl.ds(base + u * 16, 16)] = zeros
        return carry
      lax.fori_loop(0, jnp.where(active, _NBIN // _U, 0), zb, 0)

      def pb(i, carry):
        base = i * 16 * _U
        for u in range(_U):
          v = xv[pl.ds(base + u * 16, 16)]
          k = plsc.bitcast(v, jnp.int32)
          b = (k >> shift) & (_NBIN - 1)
          m = (k >> match_shift) == match_prefix
          plsc.addupdate_scatter(hist, [b * 16 + lane], ones, mask=m)
        return carry
      lax.fori_loop(0, jnp.where(active, _NVREG // _U, 0), pb, 0)

    def scan_hist(r, active):
      def gb(g, carry):
        cum, bg, beforeg = carry
        acc = hist[pl.ds(g * 256, 16)]
        for j in range(1, 16):
          acc = acc + hist[pl.ds(g * 256 + j * 16, 16)]
        newcum = cum + jnp.sum(acc)
        crossed = (newcum >= r) & (bg < 0)
        bg = jnp.where(crossed, g, bg)
        beforeg = jnp.where(crossed, cum, beforeg)
        return newcum, bg, beforeg
      _, bg, beforeg = lax.fori_loop(
          0, jnp.where(active, 64, 0), gb,
          (jnp.int32(0), jnp.int32(-1), jnp.int32(0)))

      def bb_(j, carry):
        cum, bb, before, at = carry
        sv = jnp.sum(hist[pl.ds((bg * 16 + j) * 16, 16)])
        newcum = cum + sv
        crossed = (newcum >= r) & (bb < 0)
        bb = jnp.where(crossed, bg * 16 + j, bb)
        before = jnp.where(crossed, cum, before)
        at = jnp.where(crossed, newcum, at)
        return newcum, bb, before, at
      _, bb, before, at = lax.fori_loop(
          0, jnp.where(active, 16, 0), bb_,
          (beforeg, jnp.int32(-1), jnp.int32(0), jnp.int32(0)))
      return bb, before, at

    with jax.named_scope("pass1"):
      zero_hist()
      for j in range(_NCHUNK):
        pltpu.make_async_copy(
            xs[0].at[pl.ds(j * _CHUNK, _CHUNK)],
            xv.at[pl.ds(j * _CHUNK, _CHUNK)],
            sems.at[j]).wait()
        hist_chunk(j, 20)
    with jax.named_scope("scan1"):
      b1, bef1, at1 = scan_hist(_R0, True)
    cnt1 = at1 - bef1
    rp = _R0 - bef1
    fast = cnt1 <= _CAP

    with jax.named_scope("compact"):
      for j in range(_CAP // 16 + 1):
        candv[pl.ds(j * 16, 16)] = inf16

      @plsc.parallel_loop(0, _N, step=16, unroll=_U, carry=zeros)
      def cntv(i, cv):
        v = xv[pl.ds(i, 16)]
        k = plsc.bitcast(v, jnp.int32)
        m = (k >> 20) == b1
        mi = jnp.where(m, 1, 0)
        pfx = plsc.cumsum(mi) - mi
        idx = jnp.minimum(cv + pfx, jnp.int32(_CAP + 15))
        plsc.store_scatter(candv, [idx], k, mask=m)
        return cv + plsc.all_reduce_population_count(m)
      del cntv

    with jax.named_scope("bisect"):
      kregs = [candv[pl.ds(j * 16, 16)] for j in range(_CAP // 16)]
      rp_v = zeros + rp

      def bit_body(t, kk):
        bit = 19 - t
        add = lax.shift_left(jnp.int32(1), bit)
        thr = kk | (add - 1)
        cnt = zeros
        for kr in kregs:
          cnt = cnt + plsc.all_reduce_population_count(kr <= thr)
        return jnp.where(cnt >= rp_v, kk, kk | add)
      kk = lax.fori_loop(0, 20, bit_body, zeros + (b1 << 20))
      key0_f = jnp.max(kk)

      cv = zeros
      mn = inf16
      for kr in kregs:
        cv = cv + jnp.where(kr <= key0_f, 1, 0)
        mn = jnp.minimum(mn, jnp.where(kr > key0_f, kr,
                                       jnp.int32(_POS_INF_BITS)))
      cnt_le_f = bef1 + jnp.sum(cv)
      nxt_in_bucket = jnp.min(mn)

    slow = jnp.logical_not(fast)
    hist_pass_dyn(10, 20, b1, slow)
    b2, bef2, _ = scan_hist(_R0 - bef1, slow)
    hist_pass_dyn(0, 10, (b1 << 10) | b2, slow)
    b3, _, at3 = scan_hist(_R0 - bef1 - bef2, slow)
    key0_s = (b1 << 20) | (b2 << 10) | b3
    cnt_le_s = bef1 + bef2 + at3

    key0 = jnp.where(fast, key0_f, key0_s)
    cnt_le = jnp.where(fast, cnt_le_f, cnt_le_s)
    is_dup = cnt_le >= _R0 + 1
    fast_in_b = fast & (at1 >= _R0 + 1)
    need_pass = jnp.logical_not(is_dup | fast_in_b)
    thr_min = jnp.where(fast, ((b1 + 1) << 20) - 1, key0_s)

    def mp(i, acc):
      acc = list(acc)
      base = i * 64
      for u in range(4):
        v = xv[pl.ds(base + u * 16, 16)]
        k = plsc.bitcast(v, jnp.int32)
        acc[u] = jnp.minimum(
            acc[u], jnp.where(k > thr_min, k, jnp.int32(_POS_INF_BITS)))
      return tuple(acc)
    a0, a1_, a2, a3 = lax.fori_loop(
        0, jnp.where(need_pass, _NVREG // 4, 0), mp,
        (inf16, inf16, inf16, inf16))
    min_above = jnp.min(jnp.minimum(jnp.minimum(a0, a1_),
                                    jnp.minimum(a2, a3)))

    floor_bits = jnp.where(
        is_dup, key0, jnp.where(fast_in_b, nxt_in_bucket, min_above))
    ceil_v = lax.bitcast_convert_type(key0, jnp.float32)
    floor_v = lax.bitcast_convert_type(floor_bits, jnp.float32)
    median = (ceil_v + floor_v) * 0.5

    rowv[...] = jnp.where(lane == 0, ceil_v,
                          jnp.where(lane == 1, median, 0.0))
    pltpu.sync_copy(rowv, inter_ref.at[s])
    plsc.subcore_barrier()

    @pl.when(s == 0)
    def _combine():
      pltpu.sync_copy(inter_ref, bufv)
      ceils = plsc.load_gather(bufv, [lane, zeros])
      meds = plsc.load_gather(bufv, [lane, zeros + 1])
      scaled = ceils * _MIN_RATIO
      smin = jnp.sum(scaled)
      meets = (meds >= smin).astype(jnp.float32)
      resc = _MIN_AMOUNT * (meds / smin) * meets
      plsc.store_scatter(outv, [lane, zeros],
                         jnp.full((16,), float(_N), jnp.float32))
      plsc.store_scatter(outv, [lane, zeros + 1], meds)
      plsc.store_scatter(outv, [lane, zeros + 2],
                         jnp.ones((16,), jnp.float32))
      plsc.store_scatter(outv, [lane, zeros + 3], resc)
      pltpu.sync_copy(outv, out_ref)


@functools.partial(
    pl.kernel,
    out_type=(jax.ShapeDtypeStruct((16, 4), jnp.float32),
              jax.ShapeDtypeStruct((16, 16), jnp.float32)),
    mesh=plsc.VectorSubcoreMesh(core_axis_name="c", subcore_axis_name="s"),
    compiler_params=pltpu.CompilerParams(needs_layout_passes=False),
    scratch_types=[
        pltpu.VMEM((_N,), jnp.float32),
        pltpu.VMEM((_NBIN * 16,), jnp.int32),
        pltpu.VMEM((_CAP + 16,), jnp.int32),
        pltpu.VMEM((16,), jnp.float32),
        pltpu.VMEM((16, 16), jnp.float32),
        pltpu.VMEM((16, 4), jnp.float32),
        pltpu.SemaphoreType.DMA((_NCHUNK,)),
    ],
)
def _allocator(*refs):
  _body(*refs)


def kernel(x0, x1, x2, x3, x4, x5, x6, x7, x8, x9, x10, x11, x12, x13, x14,
           x15):
  out, _ = _allocator(x0, x1, x2, x3, x4, x5, x6, x7, x8, x9, x10, x11, x12,
                      x13, x14, x15)
  return out


# speedup vs baseline: 2.3400x; 2.3400x over previous
import functools

import jax
import jax.numpy as jnp
from jax import lax
from jax.experimental import pallas as pl
from jax.experimental.pallas import tpu as pltpu
from jax.experimental.pallas import tpu_sc as plsc

_TOTAL_AMOUNT = 30000000.0
_MIN_AMOUNT = 1500.0
_MIN_RATIO = _MIN_AMOUNT / _TOTAL_AMOUNT
_N = 65536
_NBIN = 1024
_R0 = 32768
_POS_INF_BITS = 0x7F800000
_CAP = 8192
_U = 8
_NCHUNK = 8
_CHUNK = _N // _NCHUNK


def _body(x0, x1, x2, x3, x4, x5, x6, x7, x8, x9, x10, x11, x12, x13, x14,
          x15, out_ref, inter_ref, xv, hist, candv, candv2, rowv, bufv, outv,
          selr, sems):
  xs = (x0, x1, x2, x3, x4, x5, x6, x7, x8, x9, x10, x11, x12, x13, x14, x15)
  c = lax.axis_index("c")
  s = lax.axis_index("s")
  lane = lax.iota(jnp.int32, 16)
  ones = jnp.ones((16,), jnp.int32)
  zeros = jnp.zeros((16,), jnp.int32)
  inf16 = jnp.full((16,), _POS_INF_BITS, jnp.int32)

  @pl.when(c == 0)
  def _core0():
    with jax.named_scope("dma_start"):
      for a in range(16):
        @pl.when(s == a)
        def _load():
          for j in range(_NCHUNK):
            pltpu.make_async_copy(
                xs[a].at[pl.ds(j * _CHUNK, _CHUNK)],
                xv.at[pl.ds(j * _CHUNK, _CHUNK)],
                sems.at[j]).start()

    def zero_hist():
      @plsc.parallel_loop(0, _NBIN * 16, step=16, unroll=_U)
      def _zb(i):
        hist[pl.ds(i, 16)] = zeros

    def hist_chunk(j, shift):
      @plsc.parallel_loop(j * _CHUNK, (j + 1) * _CHUNK, step=16, unroll=_U)
      def _pb(i):
        v = xv[pl.ds(i, 16)]
        k = plsc.bitcast(v, jnp.int32)
        b = (k >> shift) & (_NBIN - 1)
        plsc.addupdate_scatter(hist, [b * 16 + lane], ones)

    def hist_pass(shift, match_shift, match_prefix):
      zero_hist()

      @plsc.parallel_loop(0, _N, step=16, unroll=_U)
      def _pb(i):
        v = xv[pl.ds(i, 16)]
        k = plsc.bitcast(v, jnp.int32)
        b = (k >> shift) & (_NBIN - 1)
        m = (k >> match_shift) == match_prefix
        plsc.addupdate_scatter(hist, [b * 16 + lane], ones, mask=m)

    def scan_hist(r):
      def gb(g, carry):
        cum, bg, beforeg = carry
        acc = hist[pl.ds(g * 256, 16)]
        for j in range(1, 16):
          acc = acc + hist[pl.ds(g * 256 + j * 16, 16)]
        newcum = cum + jnp.sum(acc)
        crossed = (newcum >= r) & (bg < 0)
        bg = jnp.where(crossed, g, bg)
        beforeg = jnp.where(crossed, cum, beforeg)
        return newcum, bg, beforeg
      _, bg, beforeg = lax.fori_loop(
          0, 64, gb, (jnp.int32(0), jnp.int32(-1), jnp.int32(0)))

      def bb_(j, carry):
        cum, bb, before, at = carry
        sv = jnp.sum(hist[pl.ds((bg * 16 + j) * 16, 16)])
        newcum = cum + sv
        crossed = (newcum >= r) & (bb < 0)
        bb = jnp.where(crossed, bg * 16 + j, bb)
        before = jnp.where(crossed, cum, before)
        at = jnp.where(crossed, newcum, at)
        return newcum, bb, before, at
      _, bb, before, at = lax.fori_loop(
          0, 16, bb_, (beforeg, jnp.int32(-1), jnp.int32(0), jnp.int32(0)))
      return bb, before, at

    with jax.named_scope("pass1"):
      zero_hist()
      for j in range(_NCHUNK):
        pltpu.make_async_copy(
            xs[0].at[pl.ds(j * _CHUNK, _CHUNK)],
            xv.at[pl.ds(j * _CHUNK, _CHUNK)],
            sems.at[j]).wait()
        hist_chunk(j, 20)
    with jax.named_scope("scan1"):
      b1, bef1, at1 = scan_hist(_R0)
    cnt1 = at1 - bef1
    rp = _R0 - bef1

    selr[...] = zeros

    @pl.when(cnt1 <= _CAP)
    def _fast():
      with jax.named_scope("compact"):
        @plsc.parallel_loop(0, _N, step=16, unroll=_U, carry=zeros)
        def cv_(i, cv):
          v = xv[pl.ds(i, 16)]
          k = plsc.bitcast(v, jnp.int32)
          m = (k >> 20) == b1
          mi = jnp.where(m, 1, 0)
          pfx = plsc.cumsum(mi) - mi
          plsc.store_scatter(candv, [cv + pfx], k, mask=m)
          return cv + plsc.all_reduce_population_count(m)
        del cv_

      with jax.named_scope("lvl2"):
        zero_hist()

        @plsc.parallel_loop(0, _CAP, step=16, unroll=_U)
        def h2(i):
          kv = candv[pl.ds(i, 16)]
          valid = (i + lane) < cnt1
          b = (kv >> 10) & (_NBIN - 1)
          plsc.addupdate_scatter(hist, [b * 16 + lane], ones, mask=valid)
        b2, bef2, at2 = scan_hist(rp)
        cnt2 = at2 - bef2
        rp2 = rp - bef2

        for j in range(17):
          candv2[pl.ds(j * 16, 16)] = inf16

        @plsc.parallel_loop(0, _CAP, step=16, unroll=_U, carry=zeros)
        def cv2_(i, cv):
          kv = candv[pl.ds(i, 16)]
          valid = (i + lane) < cnt1
          m = valid & (((kv >> 10) & (_NBIN - 1)) == b2)
          mi = jnp.where(m, 1, 0)
          pfx = plsc.cumsum(mi) - mi
          idx = jnp.minimum(cv + pfx, jnp.int32(271))
          plsc.store_scatter(candv2, [idx], kv, mask=m)
          return cv + plsc.all_reduce_population_count(m)
        del cv2_

      with jax.named_scope("bisect"):
        kregs = [candv2[pl.ds(j * 16, 16)] for j in range(16)]
        rp2_v = zeros + rp2

        def bit_body(t, kk):
          bit = 9 - t
          add = lax.shift_left(jnp.int32(1), bit)
          thr = kk | (add - 1)
          cnt = zeros
          for kr in kregs:
            cnt = cnt + plsc.all_reduce_population_count(kr <= thr)
          return jnp.where(cnt >= rp2_v, kk, kk | add)
        kk = lax.fori_loop(0, 10, bit_body,
                           zeros + ((b1 << 20) | (b2 << 10)))
        key0 = jnp.max(kk)

        @plsc.parallel_loop(0, _CAP, step=16, unroll=_U,
                            carry=(zeros, inf16))
        def sw(i, car):
          cv, mn = car
          kv = candv[pl.ds(i, 16)]
          valid = (i + lane) < cnt1
          cv = cv + jnp.where(valid & (kv <= key0), 1, 0)
          mn = jnp.minimum(mn, jnp.where(valid & (kv > key0), kv,
                                         jnp.int32(_POS_INF_BITS)))
          return cv, mn
        cvv, mnv = sw
        cnt_le = bef1 + jnp.sum(cvv)
        nxt_b1 = jnp.min(mnv)

      is_dup = cnt_le >= _R0 + 1
      in_b1 = at1 >= _R0 + 1
      floor_f = jnp.where(is_dup, key0, nxt_b1)
      selr[...] = jnp.where(lane == 0, key0,
                            jnp.where(lane == 1, floor_f,
                                      jnp.where(lane == 2, cnt2, 0)))

      @pl.when(jnp.logical_not(is_dup | in_b1))
      def _next_bucket():
        lim = (b1 + 1) << 20

        @plsc.parallel_loop(0, _N, step=64, unroll=2,
                            carry=(inf16, inf16, inf16, inf16))
        def accs(i, acc):
          acc = list(acc)
          for u in range(4):
            v = xv[pl.ds(i + u * 16, 16)]
            k = plsc.bitcast(v, jnp.int32)
            acc[u] = jnp.minimum(
                acc[u], jnp.where(k >= lim, k, jnp.int32(_POS_INF_BITS)))
          return tuple(acc)
        a0, a1_, a2, a3 = accs
        nxt = jnp.min(jnp.minimum(jnp.minimum(a0, a1_),
                                  jnp.minimum(a2, a3)))
        selr[...] = jnp.where(lane == 0, key0,
                              jnp.where(lane == 1, nxt,
                                        jnp.where(lane == 2, cnt2, 0)))

    cnt2_rt = jnp.max(jnp.where(lane == 2, selr[...], jnp.int32(-1)))

    @pl.when((cnt1 > _CAP) | (cnt2_rt > 256))
    def _slow():
      hist_pass(10, 20, b1)
      b2, bef2, _ = scan_hist(_R0 - bef1)
      hist_pass(0, 10, (b1 << 10) | b2)
      b3, _, at3 = scan_hist(_R0 - bef1 - bef2)
      key0 = (b1 << 20) | (b2 << 10) | b3
      cnt_le = bef1 + bef2 + at3
      selr[...] = zeros + key0

      @pl.when(cnt_le < _R0 + 1)
      def _next_larger():
        @plsc.parallel_loop(0, _N, step=64, unroll=2,
                            carry=(inf16, inf16, inf16, inf16))
        def accs(i, acc):
          acc = list(acc)
          for u in range(4):
            v = xv[pl.ds(i + u * 16, 16)]
            k = plsc.bitcast(v, jnp.int32)
            acc[u] = jnp.minimum(
                acc[u], jnp.where(k > key0, k, jnp.int32(_POS_INF_BITS)))
          return tuple(acc)
        a0, a1_, a2, a3 = accs
        nxt = jnp.min(jnp.minimum(jnp.minimum(a0, a1_),
                                  jnp.minimum(a2, a3)))
        selr[...] = jnp.where(lane == 0, key0, nxt)

    sel = selr[...]
    key0 = jnp.max(jnp.where(lane == 0, sel, jnp.int32(-2147483648)))
    floor_bits = jnp.max(jnp.where(lane == 1, sel, jnp.int32(-2147483648)))
    ceil_v = lax.bitcast_convert_type(key0, jnp.float32)
    floor_v = lax.bitcast_convert_type(floor_bits, jnp.float32)
    median = (ceil_v + floor_v) * 0.5

    rowv[...] = jnp.where(lane == 0, ceil_v,
                          jnp.where(lane == 1, median, 0.0))
    pltpu.sync_copy(rowv, inter_ref.at[s])
    plsc.subcore_barrier()

    @pl.when(s == 0)
    def _combine():
      pltpu.sync_copy(inter_ref, bufv)
      ceils = plsc.load_gather(bufv, [lane, zeros])
      meds = plsc.load_gather(bufv, [lane, zeros + 1])
      scaled = ceils * _MIN_RATIO
      smin = jnp.sum(scaled)
      meets = (meds >= smin).astype(jnp.float32)
      resc = _MIN_AMOUNT * (meds / smin) * meets
      plsc.store_scatter(outv, [lane, zeros],
                         jnp.full((16,), float(_N), jnp.float32))
      plsc.store_scatter(outv, [lane, zeros + 1], meds)
      plsc.store_scatter(outv, [lane, zeros + 2],
                         jnp.ones((16,), jnp.float32))
      plsc.store_scatter(outv, [lane, zeros + 3], resc)
      pltpu.sync_copy(outv, out_ref)


@functools.partial(
    pl.kernel,
    out_type=(jax.ShapeDtypeStruct((16, 4), jnp.float32),
              jax.ShapeDtypeStruct((16, 16), jnp.float32)),
    mesh=plsc.VectorSubcoreMesh(core_axis_name="c", subcore_axis_name="s"),
    compiler_params=pltpu.CompilerParams(needs_layout_passes=False),
    scratch_types=[
        pltpu.VMEM((_N,), jnp.float32),
        pltpu.VMEM((_NBIN * 16,), jnp.int32),
        pltpu.VMEM((_CAP + 16,), jnp.int32),
        pltpu.VMEM((272,), jnp.int32),
        pltpu.VMEM((16,), jnp.float32),
        pltpu.VMEM((16, 16), jnp.float32),
        pltpu.VMEM((16, 4), jnp.float32),
        pltpu.VMEM((16,), jnp.int32),
        pltpu.SemaphoreType.DMA((_NCHUNK,)),
    ],
)
def _allocator(*refs):
  _body(*refs)


def kernel(x0, x1, x2, x3, x4, x5, x6, x7, x8, x9, x10, x11, x12, x13, x14,
           x15):
  out, _ = _allocator(x0, x1, x2, x3, x4, x5, x6, x7, x8, x9, x10, x11, x12,
                      x13, x14, x15)
  return out
